# baseline (speedup 1.0000x reference)
import functools
import math

import jax
import jax.numpy as jnp
from jax import lax
from jax.experimental import pallas as pl
from jax.experimental.pallas import tpu as pltpu
from jax.experimental.pallas import tpu_sc as plsc

_L = 16
_NBUF = 4


def _build_table_transpose(V, D):
    NW = 32
    FB = (V // 128) // NW
    EXTRA0 = NW * FB
    REM = V // 128 - EXTRA0
    TAIL = V % 128
    mesh = plsc.VectorSubcoreMesh(core_axis_name="c", subcore_axis_name="s")

    @functools.partial(
        pl.kernel,
        mesh=mesh,
        out_type=jax.ShapeDtypeStruct((V * D,), jnp.float32),
        scratch_types=(
            [pltpu.VMEM((D, 128), jnp.float32) for _ in range(2)]
            + [pltpu.VMEM((128 * D,), jnp.float32) for _ in range(2)]
            + [pltpu.VMEM((D, 64), jnp.float32)]
            + [pltpu.SemaphoreType.DMA for _ in range(4)]
        ),
        compiler_params=pltpu.CompilerParams(use_tc_tiling_on_sc=True,
                                             needs_layout_passes=False),
    )
    def ka(tabT_hbm, out_hbm, a0, a1, o0, o1, atail, g0, g1, s0, s1):
        abufs, obufs, gsems, ssems = [a0, a1], [o0, o1], [g0, g1], [s0, s1]
        w = lax.axis_index("s") * 2 + lax.axis_index("c")
        base = w * FB

        iota = lax.iota(jnp.int32, _L)
        diag = [lax.bitwise_and(iota + jnp.int32(kd), jnp.int32(_L - 1))
                for kd in range(_L)]
        uvec = [(iota + jnp.int32(v0b * _L)) * D for v0b in range(8)]

        def in_copy(blk, a_b, gsem):
            return pltpu.make_async_copy(
                tabT_hbm.at[:, pl.ds(blk * 128, 128)], a_b, gsem)

        def out_copy(blk, o_b, ssem):
            return pltpu.make_async_copy(
                o_b, out_hbm.at[pl.ds(blk * 128 * D, 128 * D)], ssem)

        def transpose(a_b, o_b, nvb):
            def d0_body(d0g, c):
                d0 = d0g * _L
                for kd in range(_L):
                    rvec = diag[kd] + d0
                    vs = [plsc.load_gather(a_b, [rvec, iota + (v0b * _L)])
                          for v0b in range(nvb)]
                    for v0b in range(nvb):
                        plsc.store_scatter(o_b, [uvec[v0b] + rvec], vs[v0b])
                return c
            lax.fori_loop(0, D // _L, d0_body, 0)

        in_copy(base, abufs[0], gsems[0]).start()
        in_copy(base + 1, abufs[1], gsems[1]).start()

        def outer(kk, carry):
            for b in range(2):
                j = kk * 2 + b
                blk = base + j
                in_copy(blk, abufs[b], gsems[b]).wait()

                @pl.when(kk > 0)
                def _():
                    out_copy(blk - 2, obufs[b], ssems[b]).wait()

                transpose(abufs[b], obufs[b], 8)

                @pl.when(j + 2 < FB)
                def _():
                    in_copy(blk + 2, abufs[b], gsems[b]).start()

                out_copy(blk, obufs[b], ssems[b]).start()
            return carry

        lax.fori_loop(0, FB // 2, outer, 0)
        out_copy(base + FB - 2, obufs[0], ssems[0]).wait()
        out_copy(base + FB - 1, obufs[1], ssems[1]).wait()

        @pl.when(w < REM)
        def _():
            blk = jnp.int32(EXTRA0) + w
            in_copy(blk, abufs[0], gsems[0]).start()
            in_copy(blk, abufs[0], gsems[0]).wait()
            transpose(abufs[0], obufs[0], 8)
            out_copy(blk, obufs[0], ssems[0]).start()
            out_copy(blk, obufs[0], ssems[0]).wait()

        @pl.when(w == REM)
        def _():
            v0 = V - TAIL
            cp = pltpu.make_async_copy(tabT_hbm.at[:, pl.ds(v0, TAIL)],
                                       atail, gsems[1])
            cp.start()
            cp.wait()
            def d0_body(d0g, c):
                d0 = d0g * _L
                for kd in range(_L):
                    rvec = diag[kd] + d0
                    vs = [plsc.load_gather(atail, [rvec, iota + (v0b * _L)])
                          for v0b in range(TAIL // _L)]
                    for v0b in range(TAIL // _L):
                        plsc.store_scatter(obufs[1], [uvec[v0b] + rvec],
                                           vs[v0b])
                return c
            lax.fori_loop(0, D // _L, d0_body, 0)
            ocp = pltpu.make_async_copy(
                obufs[1].at[pl.ds(0, TAIL * D)],
                out_hbm.at[pl.ds(v0 * D, TAIL * D)], ssems[1])
            ocp.start()
            ocp.wait()

    return ka


def _build_kernel(V, D, S, BBLK, NBH):
    scale = jnp.float32(math.sqrt(D))
    DH = D // 8
    SHI = S // 8
    mesh = plsc.VectorSubcoreMesh(core_axis_name="c", subcore_axis_name="s")

    @functools.partial(
        pl.kernel,
        mesh=mesh,
        out_type=jax.ShapeDtypeStruct((S, DH, NBH, 8, BBLK), jnp.float32),
        scratch_types=(
            [pltpu.VMEM((SHI, 8, BBLK), jnp.int32)]
            + [pltpu.VMEM((BBLK, D), jnp.float32) for _ in range(_NBUF)]
            + [pltpu.VMEM((D, BBLK), jnp.float32) for _ in range(_NBUF)]
            + [pltpu.SemaphoreType.DMA for _ in range(2 * _NBUF)]
        ),
        compiler_params=pltpu.CompilerParams(use_tc_tiling_on_sc=False,
                                             needs_layout_passes=False),
    )
    def k(t4_hbm, tab_hbm, out_hbm, idx_v, *bufs):
        rows = bufs[0:_NBUF]
        tbs = bufs[_NBUF:2 * _NBUF]
        gsems = bufs[2 * _NBUF:3 * _NBUF]
        ssems = bufs[3 * _NBUF:4 * _NBUF]

        w = lax.axis_index("s") * 2 + lax.axis_index("c")
        pltpu.sync_copy(t4_hbm.at[:, w], idx_v)

        iota = lax.iota(jnp.int32, _L)
        row_ids = [iota + jnp.int32(blk * _L) for blk in range(BBLK // _L)]
        diag = [lax.bitwise_and(iota + jnp.int32(kd), jnp.int32(_L - 1))
                for kd in range(_L)]

        def idx_slice(step):
            return idx_v.at[lax.shift_right_logical(step, 3),
                            lax.bitwise_and(step, jnp.int32(7))]

        def gather_copy(step, rows_b, gsem):
            return pltpu.make_async_copy(tab_hbm.at[idx_slice(step)], rows_b,
                                         gsem)

        def transpose_scale(rows_b, tb_b):
            def d0_body(d0g, c):
                d0 = d0g * _L
                for kd in range(_L):
                    fvec = diag[kd] + d0
                    vs = [plsc.load_gather(rows_b, [row_ids[blk], fvec])
                          for blk in range(BBLK // _L)]
                    for blk in range(BBLK // _L):
                        plsc.store_scatter(tb_b, [fvec, row_ids[blk]],
                                           vs[blk] * scale)
                return c
            lax.fori_loop(0, D // _L, d0_body, 0)

        def start_stores(step, tb_b, ssem):
            for dh in range(DH):
                pltpu.async_copy(tb_b.at[pl.ds(dh * 8, 8)],
                                 out_hbm.at[step, dh, w], ssem)

        def wait_stores(step, tb_b, ssem):
            for dh in range(DH):
                pltpu.make_async_copy(tb_b.at[pl.ds(dh * 8, 8)],
                                     out_hbm.at[step, dh, w], ssem).wait()

        for b in range(_NBUF):
            gather_copy(jnp.int32(b), rows[b], gsems[b]).start()

        def outer(kk, carry):
            for b in range(_NBUF):
                step = kk * _NBUF + b
                gather_copy(step, rows[b], gsems[b]).wait()

                @pl.when(kk > 0)
                def _():
                    wait_stores(step, tbs[b], ssems[b])

                transpose_scale(rows[b], tbs[b])

                @pl.when(step + _NBUF < S)
                def _():
                    gather_copy(step + _NBUF, rows[b], gsems[b]).start()

                start_stores(step, tbs[b], ssems[b])
            return carry

        lax.fori_loop(0, S // _NBUF, outer, 0)
        for b in range(_NBUF):
            wait_stores(jnp.int32(S - _NBUF + b), tbs[b], ssems[b])

    return k


def kernel(tokens, table):
    B0, S = tokens.shape
    V, D = table.shape
    NBH = B0 // 128
    t4 = tokens.T.reshape(S // 8, 8, NBH, 128).transpose(0, 2, 1, 3)
    tab_lin = _build_table_transpose(V, D)(table.T).reshape(V, D)
    out5 = _build_kernel(V, D, S, 128, NBH)(t4, tab_lin)
    return out5.transpose(2, 4, 0, 1, 3).reshape(B0, S, D)


# speedup vs baseline: 1.1585x; 1.1585x over previous
import functools
import math

import jax
import jax.numpy as jnp
from jax import lax
from jax.experimental import pallas as pl
from jax.experimental.pallas import tpu as pltpu
from jax.experimental.pallas import tpu_sc as plsc

_L = 16
_NBUF = 4


def _build_table_transpose(V, D):
    NW = 32
    FB = (V // 128) // NW
    EXTRA0 = NW * FB
    REM = V // 128 - EXTRA0
    TAIL = V % 128
    mesh = plsc.VectorSubcoreMesh(core_axis_name="c", subcore_axis_name="s")

    @functools.partial(
        pl.kernel,
        mesh=mesh,
        out_type=jax.ShapeDtypeStruct((V // 2, 2 * D), jnp.float32),
        scratch_types=(
            [pltpu.VMEM((D, 128), jnp.float32) for _ in range(2)]
            + [pltpu.VMEM((64, 128), jnp.float32) for _ in range(2)]
            + [pltpu.VMEM((D, 64), jnp.float32)]
            + [pltpu.SemaphoreType.DMA for _ in range(4)]
        ),
        compiler_params=pltpu.CompilerParams(use_tc_tiling_on_sc=True,
                                             needs_layout_passes=False),
    )
    def ka(tabT_hbm, out_hbm, a0, a1, o0, o1, atail, g0, g1, s0, s1):
        abufs, obufs, gsems, ssems = [a0, a1], [o0, o1], [g0, g1], [s0, s1]
        w = lax.axis_index("s") * 2 + lax.axis_index("c")
        base = w * FB

        iota = lax.iota(jnp.int32, _L)
        diag = [lax.bitwise_and(iota + jnp.int32(kd), jnp.int32(_L - 1))
                for kd in range(_L)]
        uvec = [jnp.int32(v0b * 8) + lax.shift_right_logical(iota, 1)
                for v0b in range(8)]
        pvec = lax.bitwise_and(iota, jnp.int32(1)) * D

        def in_copy(blk, a_b, gsem):
            return pltpu.make_async_copy(
                tabT_hbm.at[:, pl.ds(blk * 128, 128)], a_b, gsem)

        def out_copy(blk, o_b, ssem):
            return pltpu.make_async_copy(
                o_b, out_hbm.at[pl.ds(blk * 64, 64)], ssem)

        def transpose(a_b, o_b, nvb):
            def d0_body(d0g, c):
                d0 = d0g * _L
                for kd in range(_L):
                    rvec = diag[kd] + d0
                    vs = [plsc.load_gather(a_b, [rvec, iota + (v0b * _L)])
                          for v0b in range(nvb)]
                    for v0b in range(nvb):
                        plsc.store_scatter(o_b, [uvec[v0b], pvec + rvec],
                                           vs[v0b])
                return c
            lax.fori_loop(0, D // _L, d0_body, 0)

        in_copy(base, abufs[0], gsems[0]).start()
        in_copy(base + 1, abufs[1], gsems[1]).start()

        def outer(kk, carry):
            for b in range(2):
                j = kk * 2 + b
                blk = base + j
                in_copy(blk, abufs[b], gsems[b]).wait()

                @pl.when(kk > 0)
                def _():
                    out_copy(blk - 2, obufs[b], ssems[b]).wait()

                transpose(abufs[b], obufs[b], 8)

                @pl.when(j + 2 < FB)
                def _():
                    in_copy(blk + 2, abufs[b], gsems[b]).start()

                out_copy(blk, obufs[b], ssems[b]).start()
            return carry

        lax.fori_loop(0, FB // 2, outer, 0)
        out_copy(base + FB - 2, obufs[0], ssems[0]).wait()
        out_copy(base + FB - 1, obufs[1], ssems[1]).wait()

        @pl.when(w < REM)
        def _():
            blk = jnp.int32(EXTRA0) + w
            in_copy(blk, abufs[0], gsems[0]).start()
            in_copy(blk, abufs[0], gsems[0]).wait()
            transpose(abufs[0], obufs[0], 8)
            out_copy(blk, obufs[0], ssems[0]).start()
            out_copy(blk, obufs[0], ssems[0]).wait()

        @pl.when(w == REM)
        def _():
            v0 = V - TAIL
            cp = pltpu.make_async_copy(tabT_hbm.at[:, pl.ds(v0, TAIL)],
                                       atail, gsems[1])
            cp.start()
            cp.wait()
            def d0_body(d0g, c):
                d0 = d0g * _L
                for kd in range(_L):
                    rvec = diag[kd] + d0
                    vs = [plsc.load_gather(atail, [rvec, iota + (v0b * _L)])
                          for v0b in range(TAIL // _L)]
                    for v0b in range(TAIL // _L):
                        plsc.store_scatter(obufs[1], [uvec[v0b], pvec + rvec],
                                           vs[v0b])
                return c
            lax.fori_loop(0, D // _L, d0_body, 0)
            ocp = pltpu.make_async_copy(
                obufs[1].at[pl.ds(0, TAIL // 2)],
                out_hbm.at[pl.ds(v0 // 2, TAIL // 2)], ssems[1])
            ocp.start()
            ocp.wait()

    return ka


def _build_kernel(V, D, S, BBLK, NBH):
    scale = jnp.float32(math.sqrt(D))
    DH = D // 8
    SHI = S // 8
    mesh = plsc.VectorSubcoreMesh(core_axis_name="c", subcore_axis_name="s")

    @functools.partial(
        pl.kernel,
        mesh=mesh,
        out_type=jax.ShapeDtypeStruct((S, DH, NBH, 8, BBLK), jnp.float32),
        scratch_types=(
            [pltpu.VMEM((SHI, 8, BBLK), jnp.int32)]
            + [pltpu.VMEM((BBLK, 2 * D), jnp.float32) for _ in range(_NBUF)]
            + [pltpu.VMEM((D, BBLK), jnp.float32) for _ in range(_NBUF)]
            + [pltpu.VMEM((BBLK,), jnp.int32) for _ in range(_NBUF)]
            + [pltpu.SemaphoreType.DMA for _ in range(2 * _NBUF)]
        ),
        compiler_params=pltpu.CompilerParams(use_tc_tiling_on_sc=True,
                                             needs_layout_passes=False),
    )
    def k(t4_hbm, tab2_hbm, out_hbm, idx_v, *bufs):
        rows = bufs[0:_NBUF]
        tbs = bufs[_NBUF:2 * _NBUF]
        i2s = bufs[2 * _NBUF:3 * _NBUF]
        gsems = bufs[3 * _NBUF:4 * _NBUF]
        ssems = bufs[4 * _NBUF:5 * _NBUF]

        w = lax.axis_index("s") * 2 + lax.axis_index("c")
        pltpu.sync_copy(t4_hbm.at[:, w], idx_v)

        iota = lax.iota(jnp.int32, _L)
        row_ids = [iota + jnp.int32(blk * _L) for blk in range(BBLK // _L)]
        diag = [lax.bitwise_and(iota + jnp.int32(kd), jnp.int32(_L - 1))
                for kd in range(_L)]

        def raw_vec(step, blk):
            return idx_v[lax.shift_right_logical(step, 3),
                         lax.bitwise_and(step, jnp.int32(7)),
                         pl.ds(blk * _L, _L)]

        def fill_idx2(step, i2_b):
            for blk in range(BBLK // _L):
                i2_b[pl.ds(blk * _L, _L)] = lax.shift_right_logical(
                    raw_vec(step, blk), 1)

        def gather_copy(rows_b, i2_b, gsem):
            return pltpu.make_async_copy(tab2_hbm.at[i2_b], rows_b, gsem)

        def transpose_scale(step, rows_b, tb_b):
            par64 = [lax.bitwise_and(raw_vec(step, blk), jnp.int32(1)) * D
                     for blk in range(BBLK // _L)]

            def d0_body(d0g, c):
                d0 = d0g * _L
                for kd in range(_L):
                    fvec = diag[kd] + d0
                    vs = [plsc.load_gather(rows_b,
                                           [row_ids[blk], fvec + par64[blk]])
                          for blk in range(BBLK // _L)]
                    for blk in range(BBLK // _L):
                        plsc.store_scatter(tb_b, [fvec, row_ids[blk]],
                                           vs[blk] * scale)
                return c
            lax.fori_loop(0, D // _L, d0_body, 0)

        def start_stores(step, tb_b, ssem):
            for dh in range(DH):
                pltpu.async_copy(tb_b.at[pl.ds(dh * 8, 8)],
                                 out_hbm.at[step, dh, w], ssem)

        def wait_stores(step, tb_b, ssem):
            for dh in range(DH):
                pltpu.make_async_copy(tb_b.at[pl.ds(dh * 8, 8)],
                                     out_hbm.at[step, dh, w], ssem).wait()

        for b in range(_NBUF):
            fill_idx2(jnp.int32(b), i2s[b])
            gather_copy(rows[b], i2s[b], gsems[b]).start()

        def outer(kk, carry):
            for b in range(_NBUF):
                step = kk * _NBUF + b
                gather_copy(rows[b], i2s[b], gsems[b]).wait()

                @pl.when(kk > 0)
                def _():
                    wait_stores(step, tbs[b], ssems[b])

                transpose_scale(step, rows[b], tbs[b])

                @pl.when(step + _NBUF < S)
                def _():
                    fill_idx2(step + _NBUF, i2s[b])
                    gather_copy(rows[b], i2s[b], gsems[b]).start()

                start_stores(step, tbs[b], ssems[b])
            return carry

        lax.fori_loop(0, S // _NBUF, outer, 0)
        for b in range(_NBUF):
            wait_stores(jnp.int32(S - _NBUF + b), tbs[b], ssems[b])

    return k


def kernel(tokens, table):
    B0, S = tokens.shape
    V, D = table.shape
    NBH = B0 // 128
    t4 = tokens.T.reshape(S // 8, 8, NBH, 128).transpose(0, 2, 1, 3)
    tab2 = _build_table_transpose(V, D)(table.T)
    out5 = _build_kernel(V, D, S, 128, NBH)(t4, tab2)
    return out5.transpose(2, 4, 0, 1, 3).reshape(B0, S, D)
